# baseline (speedup 1.0000x reference)
import functools

import jax
import jax.numpy as jnp
from jax import lax
from jax.experimental import pallas as pl
from jax.experimental.pallas import tpu as pltpu
from jax.experimental.pallas import tpu_sc as plsc

N_NODES = 10000
N_EDGES = 160000
D = 32
F = 128
DD = D * D

NW = 32
EDGES_PER_W = N_EDGES // NW
CHUNK = 40
NCHUNK = EDGES_PER_W // CHUNK
N_PAD = 10240
NODES_PER_TILE = N_PAD // 16
NCHUNK_N = NODES_PER_TILE // CHUNK


def _stats_kernel(edge_ref, w_ref, gamma_ref, beta_ref,
                  c_ref, rsum_ref, wfold_ref, t_ref):
  i = pl.program_id(0)
  n = pl.num_programs(0)
  tile = edge_ref[...]

  @pl.when(i == 0)
  def _init():
    c_ref[...] = jnp.zeros_like(c_ref)
    rsum_ref[...] = jnp.zeros_like(rsum_ref)

  c_ref[...] += lax.dot_general(
      tile, tile, (((0,), (0,)), ((), ())),
      preferred_element_type=jnp.float32)
  rsum_ref[...] += jnp.sum(
      tile.reshape(tile.shape[0] // 8, 8, F), axis=0)

  @pl.when(i == n - 1)
  def _finish():
    w = w_ref[...]
    inv_e = 1.0 / N_EDGES
    r = jnp.sum(rsum_ref[...], axis=0, keepdims=True) * inv_e
    mu = jnp.dot(r, w, preferred_element_type=jnp.float32)
    cw = jnp.dot(c_ref[...], w, preferred_element_type=jnp.float32)
    ssq = jnp.sum(w * cw, axis=0, keepdims=True) * inv_e
    var = ssq - mu * mu
    s = gamma_ref[0:1, :] * lax.rsqrt(var + 1e-5)
    wfold_ref[...] = w * s
    t_ref[...] = jnp.broadcast_to(beta_ref[0:1, :] - mu * s, t_ref.shape)


def _stats_call(edge, w_edge, gamma8, beta8):
  be = 2000
  return pl.pallas_call(
      _stats_kernel,
      grid=(N_EDGES // be,),
      in_specs=[
          pl.BlockSpec((be, F), lambda i: (i, 0)),
          pl.BlockSpec((F, DD), lambda i: (0, 0)),
          pl.BlockSpec((8, DD), lambda i: (0, 0)),
          pl.BlockSpec((8, DD), lambda i: (0, 0)),
      ],
      out_specs=[
          pl.BlockSpec((F, F), lambda i: (0, 0)),
          pl.BlockSpec((8, F), lambda i: (0, 0)),
          pl.BlockSpec((F, DD), lambda i: (0, 0)),
          pl.BlockSpec((8, DD), lambda i: (0, 0)),
      ],
      out_shape=[
          jax.ShapeDtypeStruct((F, F), jnp.float32),
          jax.ShapeDtypeStruct((8, F), jnp.float32),
          jax.ShapeDtypeStruct((F, DD), jnp.float32),
          jax.ShapeDtypeStruct((8, DD), jnp.float32),
      ],
  )(edge, w_edge, gamma8, beta8)


def _wm_kernel(edge_ref, wfold_ref, t_ref, out_ref):
  out_ref[...] = (
      jnp.dot(edge_ref[...].astype(jnp.bfloat16),
              wfold_ref[...].astype(jnp.bfloat16),
              preferred_element_type=jnp.float32)
      + t_ref[0:1, :]).astype(jnp.bfloat16)


def _wm_call(edge, wfold, t8):
  be = 1000
  return pl.pallas_call(
      _wm_kernel,
      grid=(N_EDGES // be,),
      in_specs=[
          pl.BlockSpec((be, F), lambda i: (i, 0)),
          pl.BlockSpec((F, DD), lambda i: (0, 0)),
          pl.BlockSpec((8, DD), lambda i: (0, 0)),
      ],
      out_specs=pl.BlockSpec((be, DD), lambda i: (i, 0)),
      out_shape=jax.ShapeDtypeStruct((N_EDGES, DD), jnp.bfloat16),
  )(edge, wfold, t8)


def _msg_kernel(wm_ref, xs_ref, b_ref, s_ref, out_ref):
  xe = jnp.dot(xs_ref[:, 0:D].astype(jnp.bfloat16), b_ref[...],
               preferred_element_type=jnp.float32)
  p = wm_ref[...] * xe.astype(jnp.bfloat16)
  out_ref[...] = jnp.dot(p, s_ref[...], preferred_element_type=jnp.float32)


def _msg_call(wm, xs, b_exp, s_fold):
  be = 1000
  return pl.pallas_call(
      _msg_kernel,
      grid=(N_EDGES // be,),
      in_specs=[
          pl.BlockSpec((be, DD), lambda i: (i, 0)),
          pl.BlockSpec((be, F), lambda i: (i, 0)),
          pl.BlockSpec((D, DD), lambda i: (0, 0)),
          pl.BlockSpec((DD, F), lambda i: (0, 0)),
      ],
      out_specs=pl.BlockSpec((be, F), lambda i: (i, 0)),
      out_shape=jax.ShapeDtypeStruct((N_EDGES, F), jnp.float32),
  )(wm, xs, b_exp, s_fold)


def _gru_core(aggp, cntp, h, bias1, wih, whh, bih1, bhh1):
  cnt = jnp.maximum(cntp[0][:, 0:D] + cntp[1][:, 0:D], 1.0)
  u = jax.nn.relu((aggp[0][:, 0:D] + aggp[1][:, 0:D]) / cnt + bias1)
  gx = jnp.dot(u, wih, preferred_element_type=jnp.float32) + bih1
  gh = jnp.dot(h, whh, preferred_element_type=jnp.float32) + bhh1
  r = jax.nn.sigmoid(gx[:, 0:D] + gh[:, 0:D])
  z = jax.nn.sigmoid(gx[:, D:2 * D] + gh[:, D:2 * D])
  n = jnp.tanh(gx[:, 2 * D:] + r * gh[:, 2 * D:])
  return (1.0 - z) * n + z * h


def _gru_kernel(agg_ref, cnt_ref, h_ref, acc_ref, bias_ref,
                wih_ref, whh_ref, bih_ref, bhh_ref,
                h_out_ref, acc_out_ref):
  h_new = _gru_core(agg_ref[...], cnt_ref[...], h_ref[:, 0:D],
                    bias_ref[0:1, :], wih_ref[...], whh_ref[...],
                    bih_ref[0:1, :], bhh_ref[0:1, :])
  h_out_ref[...] = jnp.concatenate([h_new] * 4, axis=1)
  acc_out_ref[...] = acc_ref[...] + h_new


def _gru_final_kernel(agg_ref, cnt_ref, h_ref, acc_ref, x0_ref, bias_ref,
                      wih_ref, whh_ref, bih_ref, bhh_ref, out_ref):
  h_new = _gru_core(agg_ref[...], cnt_ref[...], h_ref[:, 0:D],
                    bias_ref[0:1, :], wih_ref[...], whh_ref[...],
                    bih_ref[0:1, :], bhh_ref[0:1, :])
  x0 = x0_ref[...]
  out_ref[...] = x0 + 0.25 * (x0 + acc_ref[...] + h_new)


def _gru_specs(n_wide, n_narrow):
  bn = 2000
  specs = [pl.BlockSpec((2, bn, F), lambda i: (0, i, 0)),
           pl.BlockSpec((2, bn, F), lambda i: (0, i, 0))]
  specs += [pl.BlockSpec((bn, F), lambda i: (i, 0))] * n_wide
  specs += [pl.BlockSpec((bn, D), lambda i: (i, 0))] * n_narrow
  specs += [
      pl.BlockSpec((8, D), lambda i: (0, 0)),
      pl.BlockSpec((D, 3 * D), lambda i: (0, 0)),
      pl.BlockSpec((D, 3 * D), lambda i: (0, 0)),
      pl.BlockSpec((8, 3 * D), lambda i: (0, 0)),
      pl.BlockSpec((8, 3 * D), lambda i: (0, 0)),
  ]
  return bn, specs


def _gru_call(aggp, cntp, h128, acc, bias8, wiht, whht, bih8, bhh8):
  bn, specs = _gru_specs(1, 1)
  return pl.pallas_call(
      _gru_kernel,
      grid=(N_NODES // bn,),
      in_specs=specs,
      out_specs=[pl.BlockSpec((bn, F), lambda i: (i, 0)),
                 pl.BlockSpec((bn, D), lambda i: (i, 0))],
      out_shape=[jax.ShapeDtypeStruct((N_NODES, F), jnp.float32),
                 jax.ShapeDtypeStruct((N_NODES, D), jnp.float32)],
  )(aggp, cntp, h128, acc, bias8, wiht, whht, bih8, bhh8)


def _gru_final_call(aggp, cntp, h128, acc, x0, bias8, wiht, whht, bih8, bhh8):
  bn, specs = _gru_specs(1, 2)
  return pl.pallas_call(
      _gru_final_kernel,
      grid=(N_NODES // bn,),
      in_specs=specs,
      out_specs=pl.BlockSpec((bn, D), lambda i: (i, 0)),
      out_shape=jax.ShapeDtypeStruct((N_NODES, D), jnp.float32),
  )(aggp, cntp, h128, acc, x0, bias8, wiht, whht, bih8, bhh8)


def _gather_body(table_hbm, idx_hbm, out_hbm, idx_v, rows_v):
  cid = lax.axis_index("c")
  sid = lax.axis_index("s")
  wid = sid * 2 + cid
  pltpu.sync_copy(idx_hbm.at[wid], idx_v)

  def chunk(j, _):
    pltpu.sync_copy(table_hbm.at[idx_v.at[j]], rows_v)
    pltpu.sync_copy(rows_v, out_hbm.at[pl.ds(wid * EDGES_PER_W + j * CHUNK,
                                             CHUNK)])
    return 0

  lax.fori_loop(0, NCHUNK, chunk, 0)


@functools.cache
def _build_sc_kernels():
  mesh = plsc.VectorSubcoreMesh(core_axis_name="c", subcore_axis_name="s",
                                num_cores=2, num_subcores=16)
  gather = pl.kernel(
      _gather_body,
      out_type=jax.ShapeDtypeStruct((N_EDGES, F), jnp.float32),
      mesh=mesh,
      scratch_types=[
          pltpu.VMEM((NCHUNK, CHUNK), jnp.int32),
          pltpu.VMEM((CHUNK, F), jnp.float32),
      ],
  )
  scatter = pl.kernel(
      _scatter_body,
      out_type=jax.ShapeDtypeStruct((2, N_PAD, F), jnp.float32),
      mesh=mesh,
      scratch_types=[
          pltpu.VMEM((NCHUNK, CHUNK), jnp.int32),
          pltpu.VMEM((CHUNK, F), jnp.float32),
          pltpu.VMEM_SHARED((N_PAD, F), jnp.float32),
      ],
  )
  return gather, scatter


def _sc_gather(table, idx3):
  return _build_sc_kernels()[0](table, idx3)


def _scatter_body(msg_hbm, idx_hbm, zeros_hbm, out_hbm, idx_v, rows_v, agg_sh):
  cid = lax.axis_index("c")
  sid = lax.axis_index("s")
  wid = sid * 2 + cid
  base = sid * NODES_PER_TILE
  pltpu.sync_copy(idx_hbm.at[wid], idx_v)

  def zchunk(k, _):
    pltpu.sync_copy(zeros_hbm.at[pl.ds(base + k * CHUNK, CHUNK)], rows_v)
    pltpu.sync_copy(rows_v, agg_sh.at[pl.ds(base + k * CHUNK, CHUNK)])
    return 0

  lax.fori_loop(0, NCHUNK_N, zchunk, 0)
  plsc.subcore_barrier()

  def chunk(j, _):
    pltpu.sync_copy(msg_hbm.at[pl.ds(wid * EDGES_PER_W + j * CHUNK, CHUNK)],
                    rows_v)
    pltpu.sync_copy(rows_v, agg_sh.at[idx_v.at[j]], add=True)
    return 0

  lax.fori_loop(0, NCHUNK, chunk, 0)
  plsc.subcore_barrier()

  def ochunk(k, _):
    pltpu.sync_copy(agg_sh.at[pl.ds(base + k * CHUNK, CHUNK)], rows_v)
    pltpu.sync_copy(rows_v, out_hbm.at[cid].at[pl.ds(base + k * CHUNK,
                                                     CHUNK)])
    return 0

  lax.fori_loop(0, NCHUNK_N, ochunk, 0)


def _sc_scatter_add(msg, idx3, zeros_n):
  return _build_sc_kernels()[1](msg, idx3, zeros_n)


def kernel(node, edge_index, edge, W_edge, bn_gamma, bn_beta,
           W_ih, W_hh, b_ih, b_hh, bias):
  f32 = jnp.float32
  node = node.astype(f32)
  edge = edge.astype(f32)

  gamma8 = jnp.broadcast_to(bn_gamma[None, :], (8, DD)).astype(f32)
  beta8 = jnp.broadcast_to(bn_beta[None, :], (8, DD)).astype(f32)
  bias8 = jnp.broadcast_to(bias[None, :], (8, D)).astype(f32)
  wiht = W_ih.T.astype(f32)
  whht = W_hh.T.astype(f32)
  bih8 = jnp.broadcast_to(b_ih[None, :], (8, 3 * D)).astype(f32)
  bhh8 = jnp.broadcast_to(b_hh[None, :], (8, 3 * D)).astype(f32)
  src3 = edge_index[0].reshape(NW, NCHUNK, CHUNK).astype(jnp.int32)
  dst3 = edge_index[1].reshape(NW, NCHUNK, CHUNK).astype(jnp.int32)
  zeros_n = jnp.zeros((N_PAD, F), f32)
  lane = jnp.arange(DD, dtype=jnp.int32)
  b_exp = (lane[None, :] // D == jnp.arange(D, dtype=jnp.int32)[:, None]
           ).astype(jnp.bfloat16)
  s_fold = (lane[:, None] % D == (jnp.arange(F, dtype=jnp.int32) % D)[None, :]
            ).astype(jnp.bfloat16)
  zeros_acc = jnp.zeros((N_NODES, D), f32)
  ones_e = jnp.ones((N_EDGES, F), f32)

  _, _, wfold, t8 = _stats_call(edge, W_edge.astype(f32), gamma8, beta8)
  wm = _wm_call(edge, wfold, t8)

  cntp = _sc_scatter_add(ones_e, dst3, zeros_n)

  h128 = jnp.tile(node, (1, 4))
  acc = zeros_acc
  for step in range(3):
    xs = _sc_gather(h128, src3)
    msg = _msg_call(wm, xs, b_exp, s_fold)
    aggp = _sc_scatter_add(msg, dst3, zeros_n)
    if step < 2:
      h128, acc = _gru_call(aggp, cntp, h128, acc,
                            bias8, wiht, whht, bih8, bhh8)
    else:
      out = _gru_final_call(aggp, cntp, h128, acc, node,
                            bias8, wiht, whht, bih8, bhh8)
  return out


# speedup vs baseline: 1.1064x; 1.1064x over previous
import functools

import jax
import jax.numpy as jnp
from jax import lax
from jax.experimental import pallas as pl
from jax.experimental.pallas import tpu as pltpu
from jax.experimental.pallas import tpu_sc as plsc

N_NODES = 10000
N_EDGES = 160000
D = 32
F = 128
DD = D * D

NW = 32
EDGES_PER_W = N_EDGES // NW
CHUNK = 40
NCHUNK = EDGES_PER_W // CHUNK
N_PAD = 10240
NODES_PER_TILE = N_PAD // 16
NCHUNK_N = NODES_PER_TILE // CHUNK


def _stats_kernel(edge_ref, w_ref, gamma_ref, beta_ref,
                  c_ref, rsum_ref, wfold_ref, t_ref):
  i = pl.program_id(0)
  n = pl.num_programs(0)
  tile = edge_ref[...]

  @pl.when(i == 0)
  def _init():
    c_ref[...] = jnp.zeros_like(c_ref)
    rsum_ref[...] = jnp.zeros_like(rsum_ref)

  c_ref[...] += lax.dot_general(
      tile, tile, (((0,), (0,)), ((), ())),
      preferred_element_type=jnp.float32)
  rsum_ref[...] += jnp.sum(
      tile.reshape(tile.shape[0] // 8, 8, F), axis=0)

  @pl.when(i == n - 1)
  def _finish():
    w = w_ref[...]
    inv_e = 1.0 / N_EDGES
    r = jnp.sum(rsum_ref[...], axis=0, keepdims=True) * inv_e
    mu = jnp.dot(r, w, preferred_element_type=jnp.float32)
    cw = jnp.dot(c_ref[...], w, preferred_element_type=jnp.float32)
    ssq = jnp.sum(w * cw, axis=0, keepdims=True) * inv_e
    var = ssq - mu * mu
    s = gamma_ref[0:1, :] * lax.rsqrt(var + 1e-5)
    wfold_ref[...] = w * s
    t_ref[...] = jnp.broadcast_to(beta_ref[0:1, :] - mu * s, t_ref.shape)


def _stats_call(edge, w_edge, gamma8, beta8):
  be = 2000
  return pl.pallas_call(
      _stats_kernel,
      grid=(N_EDGES // be,),
      in_specs=[
          pl.BlockSpec((be, F), lambda i: (i, 0)),
          pl.BlockSpec((F, DD), lambda i: (0, 0)),
          pl.BlockSpec((8, DD), lambda i: (0, 0)),
          pl.BlockSpec((8, DD), lambda i: (0, 0)),
      ],
      out_specs=[
          pl.BlockSpec((F, F), lambda i: (0, 0)),
          pl.BlockSpec((8, F), lambda i: (0, 0)),
          pl.BlockSpec((F, DD), lambda i: (0, 0)),
          pl.BlockSpec((8, DD), lambda i: (0, 0)),
      ],
      out_shape=[
          jax.ShapeDtypeStruct((F, F), jnp.float32),
          jax.ShapeDtypeStruct((8, F), jnp.float32),
          jax.ShapeDtypeStruct((F, DD), jnp.float32),
          jax.ShapeDtypeStruct((8, DD), jnp.float32),
      ],
  )(edge, w_edge, gamma8, beta8)


def _wm_kernel(edge_ref, wfold_ref, t_ref, out_ref):
  out_ref[...] = (
      jnp.dot(edge_ref[...].astype(jnp.bfloat16),
              wfold_ref[...].astype(jnp.bfloat16),
              preferred_element_type=jnp.float32)
      + t_ref[0:1, :]).astype(jnp.bfloat16)


def _wm_call(edge, wfold, t8):
  be = 2000
  return pl.pallas_call(
      _wm_kernel,
      grid=(N_EDGES // be,),
      in_specs=[
          pl.BlockSpec((be, F), lambda i: (i, 0)),
          pl.BlockSpec((F, DD), lambda i: (0, 0)),
          pl.BlockSpec((8, DD), lambda i: (0, 0)),
      ],
      out_specs=pl.BlockSpec((be, DD), lambda i: (i, 0)),
      out_shape=jax.ShapeDtypeStruct((N_EDGES, DD), jnp.bfloat16),
  )(edge, wfold, t8)


def _msg_kernel(wm_ref, xs_ref, b_ref, s_ref, out_ref):
  xe = jnp.dot(xs_ref[:, 0:D].astype(jnp.bfloat16), b_ref[...],
               preferred_element_type=jnp.float32)
  p = wm_ref[...] * xe.astype(jnp.bfloat16)
  out_ref[...] = jnp.dot(p, s_ref[...], preferred_element_type=jnp.float32)


def _msg_call(wm, xs, b_exp, s_fold):
  be = 2000
  return pl.pallas_call(
      _msg_kernel,
      grid=(N_EDGES // be,),
      in_specs=[
          pl.BlockSpec((be, DD), lambda i: (i, 0)),
          pl.BlockSpec((be, F), lambda i: (i, 0)),
          pl.BlockSpec((D, DD), lambda i: (0, 0)),
          pl.BlockSpec((DD, F), lambda i: (0, 0)),
      ],
      out_specs=pl.BlockSpec((be, F), lambda i: (i, 0)),
      out_shape=jax.ShapeDtypeStruct((N_EDGES, F), jnp.float32),
  )(wm, xs, b_exp, s_fold)


def _gru_core(aggp, cntp, h, bias1, wih, whh, bih1, bhh1):
  cnt = jnp.maximum(cntp[0][:, 0:D] + cntp[1][:, 0:D], 1.0)
  u = jax.nn.relu((aggp[0][:, 0:D] + aggp[1][:, 0:D]) / cnt + bias1)
  gx = jnp.dot(u, wih, preferred_element_type=jnp.float32) + bih1
  gh = jnp.dot(h, whh, preferred_element_type=jnp.float32) + bhh1
  r = jax.nn.sigmoid(gx[:, 0:D] + gh[:, 0:D])
  z = jax.nn.sigmoid(gx[:, D:2 * D] + gh[:, D:2 * D])
  n = jnp.tanh(gx[:, 2 * D:] + r * gh[:, 2 * D:])
  return (1.0 - z) * n + z * h


def _gru_kernel(agg_ref, cnt_ref, h_ref, acc_ref, bias_ref,
                wih_ref, whh_ref, bih_ref, bhh_ref,
                h_out_ref, acc_out_ref):
  h_new = _gru_core(agg_ref[...], cnt_ref[...], h_ref[:, 0:D],
                    bias_ref[0:1, :], wih_ref[...], whh_ref[...],
                    bih_ref[0:1, :], bhh_ref[0:1, :])
  h_out_ref[...] = jnp.concatenate([h_new] * 4, axis=1)
  acc_out_ref[...] = acc_ref[...] + h_new


def _gru_final_kernel(agg_ref, cnt_ref, h_ref, acc_ref, x0_ref, bias_ref,
                      wih_ref, whh_ref, bih_ref, bhh_ref, out_ref):
  h_new = _gru_core(agg_ref[...], cnt_ref[...], h_ref[:, 0:D],
                    bias_ref[0:1, :], wih_ref[...], whh_ref[...],
                    bih_ref[0:1, :], bhh_ref[0:1, :])
  x0 = x0_ref[...]
  out_ref[...] = x0 + 0.25 * (x0 + acc_ref[...] + h_new)


def _gru_specs(n_wide, n_narrow):
  bn = 2000
  specs = [pl.BlockSpec((2, bn, F), lambda i: (0, i, 0)),
           pl.BlockSpec((2, bn, F), lambda i: (0, i, 0))]
  specs += [pl.BlockSpec((bn, F), lambda i: (i, 0))] * n_wide
  specs += [pl.BlockSpec((bn, D), lambda i: (i, 0))] * n_narrow
  specs += [
      pl.BlockSpec((8, D), lambda i: (0, 0)),
      pl.BlockSpec((D, 3 * D), lambda i: (0, 0)),
      pl.BlockSpec((D, 3 * D), lambda i: (0, 0)),
      pl.BlockSpec((8, 3 * D), lambda i: (0, 0)),
      pl.BlockSpec((8, 3 * D), lambda i: (0, 0)),
  ]
  return bn, specs


def _gru_call(aggp, cntp, h128, acc, bias8, wiht, whht, bih8, bhh8):
  bn, specs = _gru_specs(1, 1)
  return pl.pallas_call(
      _gru_kernel,
      grid=(N_NODES // bn,),
      in_specs=specs,
      out_specs=[pl.BlockSpec((bn, F), lambda i: (i, 0)),
                 pl.BlockSpec((bn, D), lambda i: (i, 0))],
      out_shape=[jax.ShapeDtypeStruct((N_NODES, F), jnp.float32),
                 jax.ShapeDtypeStruct((N_NODES, D), jnp.float32)],
  )(aggp, cntp, h128, acc, bias8, wiht, whht, bih8, bhh8)


def _gru_final_call(aggp, cntp, h128, acc, x0, bias8, wiht, whht, bih8, bhh8):
  bn, specs = _gru_specs(1, 2)
  return pl.pallas_call(
      _gru_final_kernel,
      grid=(N_NODES // bn,),
      in_specs=specs,
      out_specs=pl.BlockSpec((bn, D), lambda i: (i, 0)),
      out_shape=jax.ShapeDtypeStruct((N_NODES, D), jnp.float32),
  )(aggp, cntp, h128, acc, x0, bias8, wiht, whht, bih8, bhh8)


def _gather_body(table_hbm, idx_hbm, out_hbm, idx_v, rows_v):
  cid = lax.axis_index("c")
  sid = lax.axis_index("s")
  wid = sid * 2 + cid
  pltpu.sync_copy(idx_hbm.at[wid], idx_v)

  def chunk(j, _):
    pltpu.sync_copy(table_hbm.at[idx_v.at[j]], rows_v)
    pltpu.sync_copy(rows_v, out_hbm.at[pl.ds(wid * EDGES_PER_W + j * CHUNK,
                                             CHUNK)])
    return 0

  lax.fori_loop(0, NCHUNK, chunk, 0)


@functools.cache
def _build_sc_kernels():
  mesh = plsc.VectorSubcoreMesh(core_axis_name="c", subcore_axis_name="s",
                                num_cores=2, num_subcores=16)
  gather = pl.kernel(
      _gather_body,
      out_type=jax.ShapeDtypeStruct((N_EDGES, F), jnp.float32),
      mesh=mesh,
      scratch_types=[
          pltpu.VMEM((NCHUNK, CHUNK), jnp.int32),
          pltpu.VMEM((CHUNK, F), jnp.float32),
      ],
  )
  scatter = pl.kernel(
      _scatter_body,
      out_type=jax.ShapeDtypeStruct((2, N_PAD, F), jnp.float32),
      mesh=mesh,
      scratch_types=[
          pltpu.VMEM((NCHUNK, CHUNK), jnp.int32),
          pltpu.VMEM((CHUNK, F), jnp.float32),
          pltpu.VMEM_SHARED((N_PAD, F), jnp.float32),
      ],
  )
  return gather, scatter


def _sc_gather(table, idx3):
  return _build_sc_kernels()[0](table, idx3)


def _scatter_body(msg_hbm, idx_hbm, zeros_hbm, out_hbm, idx_v, rows_v, agg_sh):
  cid = lax.axis_index("c")
  sid = lax.axis_index("s")
  wid = sid * 2 + cid
  base = sid * NODES_PER_TILE
  pltpu.sync_copy(idx_hbm.at[wid], idx_v)

  def zchunk(k, _):
    pltpu.sync_copy(zeros_hbm.at[pl.ds(base + k * CHUNK, CHUNK)], rows_v)
    pltpu.sync_copy(rows_v, agg_sh.at[pl.ds(base + k * CHUNK, CHUNK)])
    return 0

  lax.fori_loop(0, NCHUNK_N, zchunk, 0)
  plsc.subcore_barrier()

  def chunk(j, _):
    pltpu.sync_copy(msg_hbm.at[pl.ds(wid * EDGES_PER_W + j * CHUNK, CHUNK)],
                    rows_v)
    pltpu.sync_copy(rows_v, agg_sh.at[idx_v.at[j]], add=True)
    return 0

  lax.fori_loop(0, NCHUNK, chunk, 0)
  plsc.subcore_barrier()

  def ochunk(k, _):
    pltpu.sync_copy(agg_sh.at[pl.ds(base + k * CHUNK, CHUNK)], rows_v)
    pltpu.sync_copy(rows_v, out_hbm.at[cid].at[pl.ds(base + k * CHUNK,
                                                     CHUNK)])
    return 0

  lax.fori_loop(0, NCHUNK_N, ochunk, 0)


def _sc_scatter_add(msg, idx3, zeros_n):
  return _build_sc_kernels()[1](msg, idx3, zeros_n)


def kernel(node, edge_index, edge, W_edge, bn_gamma, bn_beta,
           W_ih, W_hh, b_ih, b_hh, bias):
  f32 = jnp.float32
  node = node.astype(f32)
  edge = edge.astype(f32)

  gamma8 = jnp.broadcast_to(bn_gamma[None, :], (8, DD)).astype(f32)
  beta8 = jnp.broadcast_to(bn_beta[None, :], (8, DD)).astype(f32)
  bias8 = jnp.broadcast_to(bias[None, :], (8, D)).astype(f32)
  wiht = W_ih.T.astype(f32)
  whht = W_hh.T.astype(f32)
  bih8 = jnp.broadcast_to(b_ih[None, :], (8, 3 * D)).astype(f32)
  bhh8 = jnp.broadcast_to(b_hh[None, :], (8, 3 * D)).astype(f32)
  src3 = edge_index[0].reshape(NW, NCHUNK, CHUNK).astype(jnp.int32)
  dst3 = edge_index[1].reshape(NW, NCHUNK, CHUNK).astype(jnp.int32)
  zeros_n = jnp.zeros((N_PAD, F), f32)
  lane = jnp.arange(DD, dtype=jnp.int32)
  b_exp = (lane[None, :] // D == jnp.arange(D, dtype=jnp.int32)[:, None]
           ).astype(jnp.bfloat16)
  s_fold = (lane[:, None] % D == (jnp.arange(F, dtype=jnp.int32) % D)[None, :]
            ).astype(jnp.bfloat16)
  zeros_acc = jnp.zeros((N_NODES, D), f32)
  ones_e = jnp.ones((N_EDGES, F), f32)

  _, _, wfold, t8 = _stats_call(edge, W_edge.astype(f32), gamma8, beta8)
  wm = _wm_call(edge, wfold, t8)

  cntp = _sc_scatter_add(ones_e, dst3, zeros_n)

  h128 = jnp.tile(node, (1, 4))
  acc = zeros_acc
  for step in range(3):
    xs = _sc_gather(h128, src3)
    msg = _msg_call(wm, xs, b_exp, s_fold)
    aggp = _sc_scatter_add(msg, dst3, zeros_n)
    if step < 2:
      h128, acc = _gru_call(aggp, cntp, h128, acc,
                            bias8, wiht, whht, bih8, bhh8)
    else:
      out = _gru_final_call(aggp, cntp, h128, acc, node,
                            bias8, wiht, whht, bih8, bhh8)
  return out


# speedup vs baseline: 1.3415x; 1.2125x over previous
import functools

import jax
import jax.numpy as jnp
from jax import lax
from jax.experimental import pallas as pl
from jax.experimental.pallas import tpu as pltpu
from jax.experimental.pallas import tpu_sc as plsc

N_NODES = 10000
N_EDGES = 160000
D = 32
F = 128
DD = D * D

NW = 32
EDGES_PER_W = N_EDGES // NW
CHUNK = 40
NCHUNK = EDGES_PER_W // CHUNK
N_PAD = 10240
NODES_PER_TILE = N_PAD // 16
NCHUNK_N = NODES_PER_TILE // CHUNK


def _stats_kernel(edge_ref, w_ref, gamma_ref, beta_ref,
                  c_ref, rsum_ref, wfold_ref, t_ref):
  i = pl.program_id(0)
  n = pl.num_programs(0)
  tile = edge_ref[...]

  @pl.when(i == 0)
  def _init():
    c_ref[...] = jnp.zeros_like(c_ref)
    rsum_ref[...] = jnp.zeros_like(rsum_ref)

  c_ref[...] += lax.dot_general(
      tile, tile, (((0,), (0,)), ((), ())),
      preferred_element_type=jnp.float32)
  rsum_ref[...] += jnp.sum(
      tile.reshape(tile.shape[0] // 8, 8, F), axis=0)

  @pl.when(i == n - 1)
  def _finish():
    w = w_ref[...]
    inv_e = 1.0 / N_EDGES
    r = jnp.sum(rsum_ref[...], axis=0, keepdims=True) * inv_e
    mu = jnp.dot(r, w, preferred_element_type=jnp.float32)
    cw = jnp.dot(c_ref[...], w, preferred_element_type=jnp.float32)
    ssq = jnp.sum(w * cw, axis=0, keepdims=True) * inv_e
    var = ssq - mu * mu
    s = gamma_ref[0:1, :] * lax.rsqrt(var + 1e-5)
    wfold_ref[...] = w * s
    t_ref[...] = jnp.broadcast_to(beta_ref[0:1, :] - mu * s, t_ref.shape)


def _stats_call(edge, w_edge, gamma8, beta8):
  be = 2000
  return pl.pallas_call(
      _stats_kernel,
      grid=(N_EDGES // be,),
      in_specs=[
          pl.BlockSpec((be, F), lambda i: (i, 0)),
          pl.BlockSpec((F, DD), lambda i: (0, 0)),
          pl.BlockSpec((8, DD), lambda i: (0, 0)),
          pl.BlockSpec((8, DD), lambda i: (0, 0)),
      ],
      out_specs=[
          pl.BlockSpec((F, F), lambda i: (0, 0)),
          pl.BlockSpec((8, F), lambda i: (0, 0)),
          pl.BlockSpec((F, DD), lambda i: (0, 0)),
          pl.BlockSpec((8, DD), lambda i: (0, 0)),
      ],
      out_shape=[
          jax.ShapeDtypeStruct((F, F), jnp.float32),
          jax.ShapeDtypeStruct((8, F), jnp.float32),
          jax.ShapeDtypeStruct((F, DD), jnp.float32),
          jax.ShapeDtypeStruct((8, DD), jnp.float32),
      ],
  )(edge, w_edge, gamma8, beta8)


def _wm_kernel(edge_ref, wfold_ref, t_ref, out_ref):
  out_ref[...] = (
      jnp.dot(edge_ref[...].astype(jnp.bfloat16),
              wfold_ref[...].astype(jnp.bfloat16),
              preferred_element_type=jnp.float32)
      + t_ref[0:1, :]).astype(jnp.bfloat16)


def _wm_call(edge, wfold, t8):
  be = 2000
  return pl.pallas_call(
      _wm_kernel,
      grid=(N_EDGES // be,),
      in_specs=[
          pl.BlockSpec((be, F), lambda i: (i, 0)),
          pl.BlockSpec((F, DD), lambda i: (0, 0)),
          pl.BlockSpec((8, DD), lambda i: (0, 0)),
      ],
      out_specs=pl.BlockSpec((be, DD), lambda i: (i, 0)),
      out_shape=jax.ShapeDtypeStruct((N_EDGES, DD), jnp.bfloat16),
  )(edge, wfold, t8)


def _msg_kernel(wm_ref, xs_ref, b_ref, s_ref, out_ref):
  xe = jnp.dot(xs_ref[:, 0:D].astype(jnp.bfloat16), b_ref[...],
               preferred_element_type=jnp.float32)
  p = wm_ref[...] * xe.astype(jnp.bfloat16)
  out_ref[...] = jnp.dot(p, s_ref[...], preferred_element_type=jnp.float32)


def _msg_call(wm, xs, b_exp, s_fold):
  be = 2000
  return pl.pallas_call(
      _msg_kernel,
      grid=(N_EDGES // be,),
      in_specs=[
          pl.BlockSpec((be, DD), lambda i: (i, 0)),
          pl.BlockSpec((be, F), lambda i: (i, 0)),
          pl.BlockSpec((D, DD), lambda i: (0, 0)),
          pl.BlockSpec((DD, F), lambda i: (0, 0)),
      ],
      out_specs=pl.BlockSpec((be, F), lambda i: (i, 0)),
      out_shape=jax.ShapeDtypeStruct((N_EDGES, F), jnp.float32),
  )(wm, xs, b_exp, s_fold)


def _gru_core(aggp, cntp, h, bias1, wih, whh, bih1, bhh1):
  cnt = jnp.maximum(cntp[0][:, 0:D] + cntp[1][:, 0:D], 1.0)
  u = jax.nn.relu((aggp[0][:, 0:D] + aggp[1][:, 0:D]) / cnt + bias1)
  gx = jnp.dot(u, wih, preferred_element_type=jnp.float32) + bih1
  gh = jnp.dot(h, whh, preferred_element_type=jnp.float32) + bhh1
  r = jax.nn.sigmoid(gx[:, 0:D] + gh[:, 0:D])
  z = jax.nn.sigmoid(gx[:, D:2 * D] + gh[:, D:2 * D])
  n = jnp.tanh(gx[:, 2 * D:] + r * gh[:, 2 * D:])
  return (1.0 - z) * n + z * h


def _gru_kernel(agg_ref, cnt_ref, h_ref, acc_ref, bias_ref,
                wih_ref, whh_ref, bih_ref, bhh_ref,
                h_out_ref, acc_out_ref):
  h_new = _gru_core(agg_ref[...], cnt_ref[...], h_ref[:, 0:D],
                    bias_ref[0:1, :], wih_ref[...], whh_ref[...],
                    bih_ref[0:1, :], bhh_ref[0:1, :])
  h_out_ref[...] = jnp.concatenate([h_new] * 4, axis=1)
  acc_out_ref[...] = acc_ref[...] + h_new


def _gru_final_kernel(agg_ref, cnt_ref, h_ref, acc_ref, x0_ref, bias_ref,
                      wih_ref, whh_ref, bih_ref, bhh_ref, out_ref):
  h_new = _gru_core(agg_ref[...], cnt_ref[...], h_ref[:, 0:D],
                    bias_ref[0:1, :], wih_ref[...], whh_ref[...],
                    bih_ref[0:1, :], bhh_ref[0:1, :])
  x0 = x0_ref[...]
  out_ref[...] = x0 + 0.25 * (x0 + acc_ref[...] + h_new)


def _gru_specs(n_wide, n_narrow):
  bn = 2000
  specs = [pl.BlockSpec((2, bn, F), lambda i: (0, i, 0)),
           pl.BlockSpec((2, bn, F), lambda i: (0, i, 0))]
  specs += [pl.BlockSpec((bn, F), lambda i: (i, 0))] * n_wide
  specs += [pl.BlockSpec((bn, D), lambda i: (i, 0))] * n_narrow
  specs += [
      pl.BlockSpec((8, D), lambda i: (0, 0)),
      pl.BlockSpec((D, 3 * D), lambda i: (0, 0)),
      pl.BlockSpec((D, 3 * D), lambda i: (0, 0)),
      pl.BlockSpec((8, 3 * D), lambda i: (0, 0)),
      pl.BlockSpec((8, 3 * D), lambda i: (0, 0)),
  ]
  return bn, specs


def _gru_call(aggp, cntp, h128, acc, bias8, wiht, whht, bih8, bhh8):
  bn, specs = _gru_specs(1, 1)
  return pl.pallas_call(
      _gru_kernel,
      grid=(N_NODES // bn,),
      in_specs=specs,
      out_specs=[pl.BlockSpec((bn, F), lambda i: (i, 0)),
                 pl.BlockSpec((bn, D), lambda i: (i, 0))],
      out_shape=[jax.ShapeDtypeStruct((N_NODES, F), jnp.float32),
                 jax.ShapeDtypeStruct((N_NODES, D), jnp.float32)],
  )(aggp, cntp, h128, acc, bias8, wiht, whht, bih8, bhh8)


def _gru_final_call(aggp, cntp, h128, acc, x0, bias8, wiht, whht, bih8, bhh8):
  bn, specs = _gru_specs(1, 2)
  return pl.pallas_call(
      _gru_final_kernel,
      grid=(N_NODES // bn,),
      in_specs=specs,
      out_specs=pl.BlockSpec((bn, D), lambda i: (i, 0)),
      out_shape=jax.ShapeDtypeStruct((N_NODES, D), jnp.float32),
  )(aggp, cntp, h128, acc, x0, bias8, wiht, whht, bih8, bhh8)


GRP = 5
NGRP = NCHUNK // GRP


def _gather_body(table_hbm, idx_hbm, out_hbm, idx_v, rows_v, sem_g, sem_s):
  cid = lax.axis_index("c")
  sid = lax.axis_index("s")
  wid = sid * 2 + cid
  pltpu.sync_copy(idx_hbm.at[wid], idx_v)

  def group(g, _):
    gets = [
        pltpu.async_copy(table_hbm.at[idx_v.at[g * GRP + b]], rows_v.at[b],
                         sem_g)
        for b in range(GRP)
    ]
    for c in gets:
      c.wait()
    puts = [
        pltpu.async_copy(
            rows_v.at[b],
            out_hbm.at[pl.ds(wid * EDGES_PER_W + (g * GRP + b) * CHUNK,
                             CHUNK)], sem_s)
        for b in range(GRP)
    ]
    for c in puts:
      c.wait()
    return 0

  lax.fori_loop(0, NGRP, group, 0)


@functools.cache
def _build_sc_kernels():
  mesh = plsc.VectorSubcoreMesh(core_axis_name="c", subcore_axis_name="s",
                                num_cores=2, num_subcores=16)
  gather = pl.kernel(
      _gather_body,
      out_type=jax.ShapeDtypeStruct((N_EDGES, F), jnp.float32),
      mesh=mesh,
      scratch_types=[
          pltpu.VMEM((NCHUNK, CHUNK), jnp.int32),
          pltpu.VMEM((GRP, CHUNK, F), jnp.float32),
          pltpu.SemaphoreType.DMA,
          pltpu.SemaphoreType.DMA,
      ],
  )
  scatter = pl.kernel(
      _scatter_body,
      out_type=jax.ShapeDtypeStruct((2, N_PAD, F), jnp.float32),
      mesh=mesh,
      scratch_types=[
          pltpu.VMEM((NCHUNK, CHUNK), jnp.int32),
          pltpu.VMEM((GRP, CHUNK, F), jnp.float32),
          pltpu.VMEM_SHARED((N_PAD, F), jnp.float32),
          pltpu.SemaphoreType.DMA,
          pltpu.SemaphoreType.DMA,
      ],
  )
  return gather, scatter


def _sc_gather(table, idx3):
  return _build_sc_kernels()[0](table, idx3)


def _scatter_body(msg_hbm, idx_hbm, zeros_hbm, out_hbm, idx_v, rows_v, agg_sh,
                  sem_g, sem_s):
  cid = lax.axis_index("c")
  sid = lax.axis_index("s")
  wid = sid * 2 + cid
  base = sid * NODES_PER_TILE
  pltpu.sync_copy(idx_hbm.at[wid], idx_v)

  def zchunk(k, _):
    pltpu.sync_copy(zeros_hbm.at[pl.ds(base + k * CHUNK, CHUNK)],
                    rows_v.at[0])
    pltpu.sync_copy(rows_v.at[0], agg_sh.at[pl.ds(base + k * CHUNK, CHUNK)])
    return 0

  lax.fori_loop(0, NCHUNK_N, zchunk, 0)
  plsc.subcore_barrier()

  def group(g, _):
    gets = [
        pltpu.async_copy(
            msg_hbm.at[pl.ds(wid * EDGES_PER_W + (g * GRP + b) * CHUNK,
                             CHUNK)], rows_v.at[b], sem_g)
        for b in range(GRP)
    ]
    for c in gets:
      c.wait()
    adds = [
        pltpu.async_copy(rows_v.at[b], agg_sh.at[idx_v.at[g * GRP + b]],
                         sem_s, add=True)
        for b in range(GRP)
    ]
    for c in adds:
      c.wait()
    return 0

  lax.fori_loop(0, NGRP, group, 0)
  plsc.subcore_barrier()

  def ochunk(k, _):
    pltpu.sync_copy(agg_sh.at[pl.ds(base + k * CHUNK, CHUNK)], rows_v.at[0])
    pltpu.sync_copy(rows_v.at[0], out_hbm.at[cid].at[pl.ds(base + k * CHUNK,
                                                           CHUNK)])
    return 0

  lax.fori_loop(0, NCHUNK_N, ochunk, 0)


def _sc_scatter_add(msg, idx3, zeros_n):
  return _build_sc_kernels()[1](msg, idx3, zeros_n)


def kernel(node, edge_index, edge, W_edge, bn_gamma, bn_beta,
           W_ih, W_hh, b_ih, b_hh, bias):
  f32 = jnp.float32
  node = node.astype(f32)
  edge = edge.astype(f32)

  gamma8 = jnp.broadcast_to(bn_gamma[None, :], (8, DD)).astype(f32)
  beta8 = jnp.broadcast_to(bn_beta[None, :], (8, DD)).astype(f32)
  bias8 = jnp.broadcast_to(bias[None, :], (8, D)).astype(f32)
  wiht = W_ih.T.astype(f32)
  whht = W_hh.T.astype(f32)
  bih8 = jnp.broadcast_to(b_ih[None, :], (8, 3 * D)).astype(f32)
  bhh8 = jnp.broadcast_to(b_hh[None, :], (8, 3 * D)).astype(f32)
  src3 = edge_index[0].reshape(NW, NCHUNK, CHUNK).astype(jnp.int32)
  dst3 = edge_index[1].reshape(NW, NCHUNK, CHUNK).astype(jnp.int32)
  zeros_n = jnp.zeros((N_PAD, F), f32)
  lane = jnp.arange(DD, dtype=jnp.int32)
  b_exp = (lane[None, :] // D == jnp.arange(D, dtype=jnp.int32)[:, None]
           ).astype(jnp.bfloat16)
  s_fold = (lane[:, None] % D == (jnp.arange(F, dtype=jnp.int32) % D)[None, :]
            ).astype(jnp.bfloat16)
  zeros_acc = jnp.zeros((N_NODES, D), f32)
  ones_e = jnp.ones((N_EDGES, F), f32)

  _, _, wfold, t8 = _stats_call(edge, W_edge.astype(f32), gamma8, beta8)
  wm = _wm_call(edge, wfold, t8)

  cntp = _sc_scatter_add(ones_e, dst3, zeros_n)

  h128 = jnp.tile(node, (1, 4))
  acc = zeros_acc
  for step in range(3):
    xs = _sc_gather(h128, src3)
    msg = _msg_call(wm, xs, b_exp, s_fold)
    aggp = _sc_scatter_add(msg, dst3, zeros_n)
    if step < 2:
      h128, acc = _gru_call(aggp, cntp, h128, acc,
                            bias8, wiht, whht, bih8, bhh8)
    else:
      out = _gru_final_call(aggp, cntp, h128, acc, node,
                            bias8, wiht, whht, bih8, bhh8)
  return out
